# baseline (speedup 1.0000x reference)
import functools

import jax
import jax.numpy as jnp
from jax import lax
from jax.experimental import pallas as pl
from jax.experimental.pallas import tpu as pltpu
from jax.experimental.pallas import tpu_sc as plsc

N = 10000
E = 320000
HID = 32
DIM = 4
D = HID * DIM
L = 3
G = 256
VOCAB = 28
VOCAB_PAD = 32

NC = 2
NS = 16
NW = NC * NS
CHUNK = 64
EPW0 = 20544
NG0 = EPW0 // (3 * CHUNK)
E_PAD = NS * EPW0
N_ACC = 10240
ROWS_PER_TILE = N_ACC // NS

BN = 1000
NB = N // BN


def _spmm_body(z_hbm, src_hbm, dst_hbm, zeros_hbm, out_hbm,
               src_all, d0, d1, d2, r0, r1, r2, acc,
               gs0, gs1, gs2, ds0, ds1, ds2):
    c = lax.axis_index("c")
    s = lax.axis_index("s")
    pltpu.sync_copy(zeros_hbm,
                    acc.at[pl.ds(s * ROWS_PER_TILE, ROWS_PER_TILE)])
    plsc.subcore_barrier()

    rows = (r0, r1, r2)
    dsts = (d0, d1, d2)
    gsems = (gs0, gs1, gs2)
    dsems = (ds0, ds1, ds2)

    def run(ebase, ngroups):
        nch = ngroups * 3

        def gstart(j, k):
            off = pl.multiple_of(j * CHUNK, 8)
            pltpu.async_copy(dst_hbm.at[pl.ds(ebase + off, CHUNK)],
                             dsts[k], dsems[k])
            pltpu.async_copy(z_hbm.at[src_all.at[pl.ds(off, CHUNK)]],
                             rows[k], gsems[k])

        def gwait(k):
            pltpu.make_async_copy(dst_hbm.at[pl.ds(ebase, CHUNK)],
                                  dsts[k], dsems[k]).wait()
            pltpu.make_async_copy(z_hbm.at[src_all.at[pl.ds(0, CHUNK)]],
                                  rows[k], gsems[k]).wait()

        def scat(k):
            pltpu.sync_copy(rows[k], acc.at[dsts[k]], add=True)

        pltpu.sync_copy(src_hbm.at[pl.ds(ebase, nch * CHUNK)],
                        src_all.at[pl.ds(0, nch * CHUNK)])
        for k in range(3):
            gstart(k, k)

        def body(g, carry):
            j = 3 * g
            for k in range(3):
                gwait(k)
                scat(k)
                gstart(j + 3 + k, k)
            return carry

        lax.fori_loop(0, ngroups - 1, body, 0)
        for k in range(3):
            gwait(k)
            scat(k)

    @pl.when(c == 0)
    def _core0():
        run(pl.multiple_of(s * EPW0, 8), NG0)

    plsc.subcore_barrier()
    pltpu.sync_copy(acc.at[pl.ds(s * ROWS_PER_TILE, ROWS_PER_TILE)],
                    out_hbm.at[c, pl.ds(s * ROWS_PER_TILE, ROWS_PER_TILE)])


_spmm = functools.partial(
    pl.kernel,
    out_type=jax.ShapeDtypeStruct((NC, N_ACC, D), jnp.float32),
    mesh=plsc.VectorSubcoreMesh(core_axis_name="c", subcore_axis_name="s"),
    scratch_types=[
        pltpu.VMEM((EPW0,), jnp.int32),
        pltpu.VMEM((CHUNK,), jnp.int32),
        pltpu.VMEM((CHUNK,), jnp.int32),
        pltpu.VMEM((CHUNK,), jnp.int32),
        pltpu.VMEM((CHUNK, D), jnp.float32),
        pltpu.VMEM((CHUNK, D), jnp.float32),
        pltpu.VMEM((CHUNK, D), jnp.float32),
        pltpu.VMEM_SHARED((N_ACC, D), jnp.float32),
        pltpu.SemaphoreType.DMA,
        pltpu.SemaphoreType.DMA,
        pltpu.SemaphoreType.DMA,
        pltpu.SemaphoreType.DMA,
        pltpu.SemaphoreType.DMA,
        pltpu.SemaphoreType.DMA,
    ],
)(_spmm_body)


def _embed_tc(x_ref, embed_ref, m_ref, z_ref):
    xb = x_ref[...][:, 0]
    onehot = (xb[:, None]
              == lax.broadcasted_iota(jnp.int32, (BN, VOCAB_PAD), 1)
              ).astype(jnp.float32)
    em = jnp.dot(embed_ref[...], m_ref[...],
                 preferred_element_type=jnp.float32)
    z_ref[...] = jnp.dot(onehot, em, preferred_element_type=jnp.float32)


def _combine_tc(p_ref, m_ref, z_ref):
    h = jnp.maximum(p_ref[0] + p_ref[1], 0.0)
    z_ref[...] = jnp.dot(h, m_ref[...], preferred_element_type=jnp.float32)


def _final_tc(p_ref, b_ref, w1_ref, b1_ref, w2t_ref, b2_ref, o_ref, y_acc):
    i = pl.program_id(0)

    @pl.when(i == 0)
    def _init():
        y_acc[...] = jnp.zeros_like(y_acc)

    h = jnp.maximum(p_ref[0] + p_ref[1], 0.0)
    bb = b_ref[0, 0, :]
    mask = (bb[None, :]
            == lax.broadcasted_iota(jnp.int32, (G, BN), 0)).astype(jnp.float32)
    y_acc[...] += jnp.dot(mask, h, preferred_element_type=jnp.float32)

    @pl.when(i == NB - 1)
    def _readout():
        y = y_acc[...]
        t = jnp.maximum(
            jnp.dot(y, w1_ref[...], preferred_element_type=jnp.float32)
            + b1_ref[...], 0.0)
        o_ref[...] = (jnp.sum(t * w2t_ref[...], axis=1) + b2_ref[0, 0])[None, :]


def kernel(x, edge_index, batch, embed, Rs, Ws, W1, b1, W2, b2):
    Ms = (jnp.transpose(Rs, (0, 2, 1))[:, :, None, :, None]
          * Ws[:, None, :, None, :]).reshape(L, D, D)
    embed_p = jnp.pad(embed, ((0, VOCAB_PAD - VOCAB), (0, 0)))

    src = jnp.concatenate([edge_index[0],
                           jnp.zeros((E_PAD - E,), jnp.int32)])
    dst = jnp.concatenate([edge_index[1],
                           jnp.full((E_PAD - E,), N, jnp.int32)])
    zeros = jnp.zeros((ROWS_PER_TILE, D), jnp.float32)
    batch3 = batch.reshape(NB, 1, BN)

    z = pl.pallas_call(
        _embed_tc,
        grid=(NB,),
        in_specs=[
            pl.BlockSpec((BN, 1), lambda i: (i, 0)),
            pl.BlockSpec((VOCAB_PAD, D), lambda i: (0, 0)),
            pl.BlockSpec((D, D), lambda i: (0, 0)),
        ],
        out_specs=pl.BlockSpec((BN, D), lambda i: (i, 0)),
        out_shape=jax.ShapeDtypeStruct((N, D), jnp.float32),
    )(x, embed_p, Ms[0])

    for i in range(L):
        p = _spmm(z, src, dst, zeros)
        if i < L - 1:
            z = pl.pallas_call(
                _combine_tc,
                grid=(NB,),
                in_specs=[
                    pl.BlockSpec((NC, BN, D), lambda j: (0, j, 0)),
                    pl.BlockSpec((D, D), lambda j: (0, 0)),
                ],
                out_specs=pl.BlockSpec((BN, D), lambda j: (j, 0)),
                out_shape=jax.ShapeDtypeStruct((N, D), jnp.float32),
            )(p, Ms[i + 1])

    out2d = pl.pallas_call(
        _final_tc,
        grid=(NB,),
        in_specs=[
            pl.BlockSpec((NC, BN, D), lambda j: (0, j, 0)),
            pl.BlockSpec((1, 1, BN), lambda j: (j, 0, 0)),
            pl.BlockSpec((D, HID), lambda j: (0, 0)),
            pl.BlockSpec((1, HID), lambda j: (0, 0)),
            pl.BlockSpec((1, HID), lambda j: (0, 0)),
            pl.BlockSpec((1, 1), lambda j: (0, 0)),
        ],
        out_specs=pl.BlockSpec((1, G), lambda j: (0, 0)),
        out_shape=jax.ShapeDtypeStruct((1, G), jnp.float32),
        scratch_shapes=[pltpu.VMEM((G, D), jnp.float32)],
    )(p, batch3, W1, b1.reshape(1, HID), W2.reshape(1, HID), b2.reshape(1, 1))

    return out2d[0]


# speedup vs baseline: 1.3729x; 1.3729x over previous
import functools

import jax
import jax.numpy as jnp
from jax import lax
from jax.experimental import pallas as pl
from jax.experimental.pallas import tpu as pltpu
from jax.experimental.pallas import tpu_sc as plsc

N = 10000
E = 320000
HID = 32
DIM = 4
D = HID * DIM
L = 3
G = 256
VOCAB = 28
VOCAB_PAD = 32

NC = 2
NS = 16
NW = NC * NS
CHUNK = 80
EPW0 = 15600
EPW1 = 4800
NG0 = EPW0 // (3 * CHUNK)
NG1 = EPW1 // (3 * CHUNK)
OFF1 = NS * EPW0
E_PAD = NS * (EPW0 + EPW1)
HCHUNK = 128
HEPW = 10240
HNCH = HEPW // HCHUNK
E_PAD_H = NW * HEPW
NX = 10240
N_ACC = 10240
ROWS_PER_TILE = N_ACC // NS
ACC32 = N_ACC * VOCAB_PAD
ACC32_PT = ACC32 // NS

BN = 1000
NB = N // BN


def _spmm_body(z_hbm, src_hbm, dst_hbm, zeros_hbm, out_hbm,
               src_all, d0, d1, d2, r0, r1, r2, acc,
               gs0, gs1, gs2, ds0, ds1, ds2):
    c = lax.axis_index("c")
    s = lax.axis_index("s")
    pltpu.sync_copy(zeros_hbm,
                    acc.at[pl.ds(s * ROWS_PER_TILE, ROWS_PER_TILE)])
    plsc.subcore_barrier()

    rows = (r0, r1, r2)
    dsts = (d0, d1, d2)
    gsems = (gs0, gs1, gs2)
    dsems = (ds0, ds1, ds2)

    def run(ebase, ngroups):
        nch = ngroups * 3

        def gstart(j, k):
            off = pl.multiple_of(j * CHUNK, 8)
            pltpu.async_copy(dst_hbm.at[pl.ds(ebase + off, CHUNK)],
                             dsts[k], dsems[k])
            pltpu.async_copy(z_hbm.at[src_all.at[pl.ds(off, CHUNK)]],
                             rows[k], gsems[k])

        def gwait(k):
            pltpu.make_async_copy(dst_hbm.at[pl.ds(ebase, CHUNK)],
                                  dsts[k], dsems[k]).wait()
            pltpu.make_async_copy(z_hbm.at[src_all.at[pl.ds(0, CHUNK)]],
                                  rows[k], gsems[k]).wait()

        def scat(k):
            pltpu.sync_copy(rows[k], acc.at[dsts[k]], add=True)

        pltpu.sync_copy(src_hbm.at[pl.ds(ebase, nch * CHUNK)],
                        src_all.at[pl.ds(0, nch * CHUNK)])
        for k in range(3):
            gstart(k, k)

        def body(g, carry):
            j = 3 * g
            for k in range(3):
                gwait(k)
                scat(k)
                gstart(j + 3 + k, k)
            return carry

        lax.fori_loop(0, ngroups - 1, body, 0)
        for k in range(3):
            gwait(k)
            scat(k)

    @pl.when(c == 0)
    def _core0():
        run(pl.multiple_of(s * EPW0, 8), NG0)

    @pl.when(c == 1)
    def _core1():
        run(pl.multiple_of(OFF1 + s * EPW1, 8), NG1)

    plsc.subcore_barrier()
    pltpu.sync_copy(acc.at[pl.ds(s * ROWS_PER_TILE, ROWS_PER_TILE)],
                    out_hbm.at[c, pl.ds(s * ROWS_PER_TILE, ROWS_PER_TILE)])


_spmm = functools.partial(
    pl.kernel,
    out_type=jax.ShapeDtypeStruct((NC, N_ACC, D), jnp.float32),
    mesh=plsc.VectorSubcoreMesh(core_axis_name="c", subcore_axis_name="s"),
    scratch_types=[
        pltpu.VMEM((EPW0,), jnp.int32),
        pltpu.VMEM((CHUNK,), jnp.int32),
        pltpu.VMEM((CHUNK,), jnp.int32),
        pltpu.VMEM((CHUNK,), jnp.int32),
        pltpu.VMEM((CHUNK, D), jnp.float32),
        pltpu.VMEM((CHUNK, D), jnp.float32),
        pltpu.VMEM((CHUNK, D), jnp.float32),
        pltpu.VMEM_SHARED((N_ACC, D), jnp.float32),
        pltpu.SemaphoreType.DMA,
        pltpu.SemaphoreType.DMA,
        pltpu.SemaphoreType.DMA,
        pltpu.SemaphoreType.DMA,
        pltpu.SemaphoreType.DMA,
        pltpu.SemaphoreType.DMA,
    ],
)(_spmm_body)


def _hist_body(x_hbm, src_hbm, dst_hbm, zeros_hbm, out_hbm,
               xsh, src_all, dst_all, xs0, xs1, idx0, idx1, ones_v, acc,
               xm0, xm1, sm0, sm1):
    c = lax.axis_index("c")
    s = lax.axis_index("s")
    wid = s * NC + c
    pltpu.sync_copy(zeros_hbm, acc.at[pl.ds(s * ACC32_PT, ACC32_PT)])

    @pl.when(s == 0)
    def _():
        pltpu.sync_copy(x_hbm, xsh)

    ebase = pl.multiple_of(wid * HEPW, 8)
    pltpu.sync_copy(src_hbm.at[pl.ds(ebase, HEPW)], src_all)
    pltpu.sync_copy(dst_hbm.at[pl.ds(ebase, HEPW)], dst_all)
    for t in range(HCHUNK // 16):
        ones_v[pl.ds(16 * t, 16)] = jnp.full((16,), 1.0, jnp.float32)
    plsc.subcore_barrier()

    xss = (xs0, xs1)
    idxs = (idx0, idx1)
    xsems = (xm0, xm1)
    ssems = (sm0, sm1)

    def xstart(j, k):
        off = pl.multiple_of(j * HCHUNK, 8)
        pltpu.async_copy(xsh.at[src_all.at[pl.ds(off, HCHUNK)]],
                         xss[k], xsems[k])

    def xwait(k):
        pltpu.make_async_copy(xsh.at[src_all.at[pl.ds(0, HCHUNK)]],
                              xss[k], xsems[k]).wait()

    def build(j, k):
        for t in range(HCHUNK // 16):
            off = j * HCHUNK + t * 16
            dv = dst_all[pl.ds(off, 16)]
            xv = xss[k][pl.ds(16 * t, 16)]
            idxs[k][pl.ds(16 * t, 16)] = dv * VOCAB_PAD + xv

    def sstart(k):
        pltpu.async_copy(ones_v, acc.at[idxs[k]], ssems[k], add=True)

    def swait(k):
        pltpu.make_async_copy(ones_v, acc.at[idxs[k]], ssems[k]).wait()

    xstart(0, 0)
    xstart(1, 1)
    xwait(0)
    build(0, 0)
    sstart(0)
    xstart(2, 0)
    xwait(1)
    build(1, 1)
    sstart(1)
    xstart(3, 1)

    def body(g, carry):
        j = 2 * g
        xwait(0)
        swait(0)
        build(j, 0)
        sstart(0)
        xstart(j + 2, 0)
        xwait(1)
        swait(1)
        build(j + 1, 1)
        sstart(1)
        xstart(j + 3, 1)
        return carry

    lax.fori_loop(1, HNCH // 2 - 1, body, 0)
    for (j, k) in ((HNCH - 2, 0), (HNCH - 1, 1)):
        xwait(k)
        swait(k)
        build(j, k)
        sstart(k)
    swait(0)
    swait(1)
    plsc.subcore_barrier()
    pltpu.sync_copy(acc.at[pl.ds(s * ACC32_PT, ACC32_PT)],
                    out_hbm.at[pl.ds(c * ACC32 + s * ACC32_PT, ACC32_PT)])


_hist = functools.partial(
    pl.kernel,
    out_type=jax.ShapeDtypeStruct((NC * ACC32,), jnp.float32),
    mesh=plsc.VectorSubcoreMesh(core_axis_name="c", subcore_axis_name="s"),
    scratch_types=[
        pltpu.VMEM_SHARED((NX,), jnp.int32),
        pltpu.VMEM((HEPW,), jnp.int32),
        pltpu.VMEM((HEPW,), jnp.int32),
        pltpu.VMEM((HCHUNK,), jnp.int32),
        pltpu.VMEM((HCHUNK,), jnp.int32),
        pltpu.VMEM((HCHUNK,), jnp.int32),
        pltpu.VMEM((HCHUNK,), jnp.int32),
        pltpu.VMEM((HCHUNK,), jnp.float32),
        pltpu.VMEM_SHARED((ACC32,), jnp.float32),
        pltpu.SemaphoreType.DMA,
        pltpu.SemaphoreType.DMA,
        pltpu.SemaphoreType.DMA,
        pltpu.SemaphoreType.DMA,
    ],
)(_hist_body)


def _combine1_tc(p_ref, embed_ref, m0_ref, m1_ref, d0, d1, d2, d3, z_ref):
    del d0, d1, d2, d3
    hist = p_ref[0] + p_ref[1]
    em = jnp.dot(embed_ref[...], m0_ref[...],
                 preferred_element_type=jnp.float32,
                 precision=lax.Precision.HIGHEST)
    h = jnp.maximum(jnp.dot(hist, em, preferred_element_type=jnp.float32,
                 precision=lax.Precision.HIGHEST),
                    0.0)
    z_ref[...] = jnp.dot(h, m1_ref[...], preferred_element_type=jnp.float32,
                 precision=lax.Precision.HIGHEST)


def _embed_tc(x_ref, embed_ref, m_ref, z_ref):
    xb = x_ref[...][:, 0]
    onehot = (xb[:, None]
              == lax.broadcasted_iota(jnp.int32, (BN, VOCAB_PAD), 1)
              ).astype(jnp.float32)
    em = jnp.dot(embed_ref[...], m_ref[...],
                 preferred_element_type=jnp.float32,
                 precision=lax.Precision.HIGHEST)
    z_ref[...] = jnp.dot(onehot, em, preferred_element_type=jnp.float32,
                 precision=lax.Precision.HIGHEST)


def _combine_tc(p_ref, m_ref, d0, z_ref):
    del d0
    h = jnp.maximum(p_ref[0] + p_ref[1], 0.0)
    z_ref[...] = jnp.dot(h, m_ref[...], preferred_element_type=jnp.float32,
                 precision=lax.Precision.HIGHEST)


def _final_tc(p_ref, b_ref, w1_ref, b1_ref, w2t_ref, b2_ref,
              d0, d1, d2, d3, d4, o_ref, y_acc):
    del d0, d1, d2, d3, d4
    i = pl.program_id(0)

    @pl.when(i == 0)
    def _init():
        y_acc[...] = jnp.zeros_like(y_acc)

    h = jnp.maximum(p_ref[0] + p_ref[1], 0.0)
    bb = b_ref[0, 0, :]
    mask = (bb[None, :]
            == lax.broadcasted_iota(jnp.int32, (G, BN), 0)).astype(jnp.float32)
    y_acc[...] += jnp.dot(mask, h, preferred_element_type=jnp.float32,
                 precision=lax.Precision.HIGHEST)

    @pl.when(i == NB - 1)
    def _readout():
        y = y_acc[...]
        t = jnp.maximum(
            jnp.dot(y, w1_ref[...], preferred_element_type=jnp.float32,
                 precision=lax.Precision.HIGHEST)
            + b1_ref[...], 0.0)
        o_ref[...] = (jnp.sum(t * w2t_ref[...], axis=1) + b2_ref[0, 0])[None, :]


def kernel(x, edge_index, batch, embed, Rs, Ws, W1, b1, W2, b2):
    Ms = (jnp.transpose(Rs, (0, 2, 1))[:, :, None, :, None]
          * Ws[:, None, :, None, :]).reshape(L, D, D)
    embed_p = jnp.pad(embed, ((0, VOCAB_PAD - VOCAB), (0, 0)))

    src = jnp.concatenate([edge_index[0],
                           jnp.zeros((E_PAD - E,), jnp.int32)])
    dst = jnp.concatenate([edge_index[1],
                           jnp.full((E_PAD - E,), N, jnp.int32)])
    zeros = jnp.zeros((ROWS_PER_TILE, D), jnp.float32)
    batch3 = batch.reshape(NB, 1, BN)

    z = pl.pallas_call(
        _embed_tc,
        grid=(NB,),
        in_specs=[
            pl.BlockSpec((BN, 1), lambda i: (i, 0)),
            pl.BlockSpec((VOCAB_PAD, D), lambda i: (0, 0)),
            pl.BlockSpec((D, D), lambda i: (0, 0)),
        ],
        out_specs=pl.BlockSpec((BN, D), lambda i: (i, 0)),
        out_shape=jax.ShapeDtypeStruct((N, D), jnp.float32),
    )(x, embed_p, Ms[0])

    p1keep = None
    for i in range(L):
        zp = z
        p = _spmm(z, src, dst, zeros)
        if p1keep is None:
            p1keep = p
        if i < L - 1:
            z = pl.pallas_call(
                _combine_tc,
                grid=(NB,),
                in_specs=[
                    pl.BlockSpec((NC, BN, D), lambda j: (0, j, 0)),
                    pl.BlockSpec((D, D), lambda j: (0, 0)),
                    pl.BlockSpec(memory_space=pl.ANY),
                ],
                out_specs=pl.BlockSpec((BN, D), lambda j: (j, 0)),
                out_shape=jax.ShapeDtypeStruct((N, D), jnp.float32),
            )(p, Ms[i + 1], zp)

    out2d = pl.pallas_call(
        _final_tc,
        grid=(NB,),
        in_specs=[
            pl.BlockSpec((NC, BN, D), lambda j: (0, j, 0)),
            pl.BlockSpec((1, 1, BN), lambda j: (j, 0, 0)),
            pl.BlockSpec((D, HID), lambda j: (0, 0)),
            pl.BlockSpec((1, HID), lambda j: (0, 0)),
            pl.BlockSpec((1, HID), lambda j: (0, 0)),
            pl.BlockSpec((1, 1), lambda j: (0, 0)),
            pl.BlockSpec(memory_space=pl.ANY),
            pl.BlockSpec(memory_space=pl.ANY),
            pl.BlockSpec(memory_space=pl.ANY),
            pl.BlockSpec(memory_space=pl.ANY),
            pl.BlockSpec(memory_space=pl.ANY),
        ],
        out_specs=pl.BlockSpec((1, G), lambda j: (0, 0)),
        out_shape=jax.ShapeDtypeStruct((1, G), jnp.float32),
        scratch_shapes=[pltpu.VMEM((G, D), jnp.float32)],
    )(p, batch3, W1, b1.reshape(1, HID), W2.reshape(1, HID), b2.reshape(1, 1),
      zp, src, dst, zeros, p1keep)

    return out2d[0]


# speedup vs baseline: 2.7554x; 2.0069x over previous
import functools

import jax
import jax.numpy as jnp
from jax import lax
from jax.experimental import pallas as pl
from jax.experimental.pallas import tpu as pltpu
from jax.experimental.pallas import tpu_sc as plsc

N = 10000
E = 320000
HID = 32
DIM = 4
D = HID * DIM
L = 3
G = 256
VOCAB = 28
VOCAB_PAD = 32

NC = 2
NS = 16
NW = NC * NS
CHUNK = 80
EPW0 = 17760
EPW1 = 2400
NG0 = EPW0 // (3 * CHUNK)
NG1 = EPW1 // (3 * CHUNK)
OFF1 = NS * EPW0
E_PAD = NS * (EPW0 + EPW1)
HCHUNK = 128
HEPW = 10240
HNCH = HEPW // HCHUNK
E_PAD_H = NW * HEPW
NX = 10240
N_ACC = 10240
ROWS_PER_TILE = N_ACC // NS
ACC32 = N_ACC * VOCAB_PAD
ACC32_PT = ACC32 // NS

BN = 1000
NB = N // BN


def _spmm_body(z_hbm, src_hbm, dst_hbm, zeros_hbm, out_hbm,
               src_all, d0, d1, d2, r0, r1, r2, acc,
               gs0, gs1, gs2, ds0, ds1, ds2):
    c = lax.axis_index("c")
    s = lax.axis_index("s")
    pltpu.sync_copy(zeros_hbm,
                    acc.at[pl.ds(s * ROWS_PER_TILE, ROWS_PER_TILE)])
    plsc.subcore_barrier()

    rows = (r0, r1, r2)
    dsts = (d0, d1, d2)
    gsems = (gs0, gs1, gs2)
    dsems = (ds0, ds1, ds2)

    def run(ebase, ngroups):
        nch = ngroups * 3

        def gstart(j, k):
            off = pl.multiple_of(j * CHUNK, 8)
            pltpu.async_copy(dst_hbm.at[pl.ds(ebase + off, CHUNK)],
                             dsts[k], dsems[k])
            pltpu.async_copy(z_hbm.at[src_all.at[pl.ds(off, CHUNK)]],
                             rows[k], gsems[k])

        def gwait(k):
            pltpu.make_async_copy(dst_hbm.at[pl.ds(ebase, CHUNK)],
                                  dsts[k], dsems[k]).wait()
            pltpu.make_async_copy(z_hbm.at[src_all.at[pl.ds(0, CHUNK)]],
                                  rows[k], gsems[k]).wait()

        def scat(k):
            pltpu.sync_copy(rows[k], acc.at[dsts[k]], add=True)

        pltpu.sync_copy(src_hbm.at[pl.ds(ebase, nch * CHUNK)],
                        src_all.at[pl.ds(0, nch * CHUNK)])
        for k in range(3):
            gstart(k, k)

        def body(g, carry):
            j = 3 * g
            for k in range(3):
                gwait(k)
                scat(k)
                gstart(j + 3 + k, k)
            return carry

        lax.fori_loop(0, ngroups - 1, body, 0)
        for k in range(3):
            gwait(k)
            scat(k)

    @pl.when(c == 0)
    def _core0():
        run(pl.multiple_of(s * EPW0, 8), NG0)

    @pl.when(c == 1)
    def _core1():
        run(pl.multiple_of(OFF1 + s * EPW1, 8), NG1)

    plsc.subcore_barrier()
    pltpu.sync_copy(acc.at[pl.ds(s * ROWS_PER_TILE, ROWS_PER_TILE)],
                    out_hbm.at[c, pl.ds(s * ROWS_PER_TILE, ROWS_PER_TILE)])


_spmm = functools.partial(
    pl.kernel,
    out_type=jax.ShapeDtypeStruct((NC, N_ACC, D), jnp.float32),
    mesh=plsc.VectorSubcoreMesh(core_axis_name="c", subcore_axis_name="s"),
    scratch_types=[
        pltpu.VMEM((EPW0,), jnp.int32),
        pltpu.VMEM((CHUNK,), jnp.int32),
        pltpu.VMEM((CHUNK,), jnp.int32),
        pltpu.VMEM((CHUNK,), jnp.int32),
        pltpu.VMEM((CHUNK, D), jnp.float32),
        pltpu.VMEM((CHUNK, D), jnp.float32),
        pltpu.VMEM((CHUNK, D), jnp.float32),
        pltpu.VMEM_SHARED((N_ACC, D), jnp.float32),
        pltpu.SemaphoreType.DMA,
        pltpu.SemaphoreType.DMA,
        pltpu.SemaphoreType.DMA,
        pltpu.SemaphoreType.DMA,
        pltpu.SemaphoreType.DMA,
        pltpu.SemaphoreType.DMA,
    ],
)(_spmm_body)


def _hist_body(x_hbm, src_hbm, dst_hbm, zeros_hbm, out_hbm,
               xsh, src_all, dst_all, xs0, xs1, idx0, idx1, ones_v, acc,
               xm0, xm1, sm0, sm1):
    c = lax.axis_index("c")
    s = lax.axis_index("s")
    wid = s * NC + c
    pltpu.sync_copy(zeros_hbm, acc.at[pl.ds(s * ACC32_PT, ACC32_PT)])

    @pl.when(s == 0)
    def _():
        pltpu.sync_copy(x_hbm, xsh)

    ebase = pl.multiple_of(wid * HEPW, 8)
    pltpu.sync_copy(src_hbm.at[pl.ds(ebase, HEPW)], src_all)
    pltpu.sync_copy(dst_hbm.at[pl.ds(ebase, HEPW)], dst_all)
    for t in range(HCHUNK // 16):
        ones_v[pl.ds(16 * t, 16)] = jnp.full((16,), 1.0, jnp.float32)
    plsc.subcore_barrier()

    xss = (xs0, xs1)
    idxs = (idx0, idx1)
    xsems = (xm0, xm1)
    ssems = (sm0, sm1)

    def xstart(j, k):
        off = pl.multiple_of(j * HCHUNK, 8)
        pltpu.async_copy(xsh.at[src_all.at[pl.ds(off, HCHUNK)]],
                         xss[k], xsems[k])

    def xwait(k):
        pltpu.make_async_copy(xsh.at[src_all.at[pl.ds(0, HCHUNK)]],
                              xss[k], xsems[k]).wait()

    def build(j, k):
        for t in range(HCHUNK // 16):
            off = j * HCHUNK + t * 16
            dv = dst_all[pl.ds(off, 16)]
            xv = xss[k][pl.ds(16 * t, 16)]
            idxs[k][pl.ds(16 * t, 16)] = dv * VOCAB_PAD + xv

    def sstart(k):
        pltpu.async_copy(ones_v, acc.at[idxs[k]], ssems[k], add=True)

    def swait(k):
        pltpu.make_async_copy(ones_v, acc.at[idxs[k]], ssems[k]).wait()

    xstart(0, 0)
    xstart(1, 1)
    xwait(0)
    build(0, 0)
    sstart(0)
    xstart(2, 0)
    xwait(1)
    build(1, 1)
    sstart(1)
    xstart(3, 1)

    def body(g, carry):
        j = 2 * g
        xwait(0)
        swait(0)
        build(j, 0)
        sstart(0)
        xstart(j + 2, 0)
        xwait(1)
        swait(1)
        build(j + 1, 1)
        sstart(1)
        xstart(j + 3, 1)
        return carry

    lax.fori_loop(1, HNCH // 2 - 1, body, 0)
    for (j, k) in ((HNCH - 2, 0), (HNCH - 1, 1)):
        xwait(k)
        swait(k)
        build(j, k)
        sstart(k)
    swait(0)
    swait(1)
    plsc.subcore_barrier()
    pltpu.sync_copy(acc.at[pl.ds(s * ACC32_PT, ACC32_PT)],
                    out_hbm.at[pl.ds(c * ACC32 + s * ACC32_PT, ACC32_PT)])


_hist = functools.partial(
    pl.kernel,
    out_type=jax.ShapeDtypeStruct((NC * ACC32,), jnp.float32),
    mesh=plsc.VectorSubcoreMesh(core_axis_name="c", subcore_axis_name="s"),
    scratch_types=[
        pltpu.VMEM_SHARED((NX,), jnp.int32),
        pltpu.VMEM((HEPW,), jnp.int32),
        pltpu.VMEM((HEPW,), jnp.int32),
        pltpu.VMEM((HCHUNK,), jnp.int32),
        pltpu.VMEM((HCHUNK,), jnp.int32),
        pltpu.VMEM((HCHUNK,), jnp.int32),
        pltpu.VMEM((HCHUNK,), jnp.int32),
        pltpu.VMEM((HCHUNK,), jnp.float32),
        pltpu.VMEM_SHARED((ACC32,), jnp.float32),
        pltpu.SemaphoreType.DMA,
        pltpu.SemaphoreType.DMA,
        pltpu.SemaphoreType.DMA,
        pltpu.SemaphoreType.DMA,
    ],
)(_hist_body)


def _combine1_tc(p_ref, embed_ref, m0_ref, m1_ref, d0, d1, d2, d3, z_ref):
    del d0, d1, d2, d3
    hist = p_ref[0] + p_ref[1]
    em = jnp.dot(embed_ref[...], m0_ref[...],
                 preferred_element_type=jnp.float32,
                 precision=lax.Precision.HIGHEST)
    h = jnp.maximum(jnp.dot(hist, em, preferred_element_type=jnp.float32,
                 precision=lax.Precision.HIGHEST),
                    0.0)
    z_ref[...] = jnp.dot(h, m1_ref[...], preferred_element_type=jnp.float32,
                 precision=lax.Precision.HIGHEST)


def _embed_tc(x_ref, embed_ref, m_ref, z_ref):
    xb = x_ref[...][:, 0]
    onehot = (xb[:, None]
              == lax.broadcasted_iota(jnp.int32, (BN, VOCAB_PAD), 1)
              ).astype(jnp.float32)
    em = jnp.dot(embed_ref[...], m_ref[...],
                 preferred_element_type=jnp.float32,
                 precision=lax.Precision.HIGHEST)
    z_ref[...] = jnp.dot(onehot, em, preferred_element_type=jnp.float32,
                 precision=lax.Precision.HIGHEST)


def _combine_tc(p_ref, m_ref, d0, z_ref):
    del d0
    h = jnp.maximum(p_ref[0] + p_ref[1], 0.0)
    z_ref[...] = jnp.dot(h, m_ref[...], preferred_element_type=jnp.float32,
                 precision=lax.Precision.HIGHEST)


def _final_tc(p_ref, b_ref, w1_ref, b1_ref, w2t_ref, b2_ref,
              d0, d1, d2, d3, d4, o_ref, y_acc):
    del d0, d1, d2, d3, d4
    i = pl.program_id(0)

    @pl.when(i == 0)
    def _init():
        y_acc[...] = jnp.zeros_like(y_acc)

    h = jnp.maximum(p_ref[0] + p_ref[1], 0.0)
    bb = b_ref[0, 0, :]
    mask = (bb[None, :]
            == lax.broadcasted_iota(jnp.int32, (G, BN), 0)).astype(jnp.float32)
    y_acc[...] += jnp.dot(mask, h, preferred_element_type=jnp.float32,
                 precision=lax.Precision.HIGHEST)

    @pl.when(i == NB - 1)
    def _readout():
        y = y_acc[...]
        t = jnp.maximum(
            jnp.dot(y, w1_ref[...], preferred_element_type=jnp.float32,
                 precision=lax.Precision.HIGHEST)
            + b1_ref[...], 0.0)
        o_ref[...] = (jnp.sum(t * w2t_ref[...], axis=1) + b2_ref[0, 0])[None, :]


def kernel(x, edge_index, batch, embed, Rs, Ws, W1, b1, W2, b2):
    Ms = (jnp.transpose(Rs, (0, 2, 1))[:, :, None, :, None]
          * Ws[:, None, :, None, :]).reshape(L, D, D)
    embed_p = jnp.pad(embed, ((0, VOCAB_PAD - VOCAB), (0, 0)))

    src = jnp.concatenate([edge_index[0],
                           jnp.zeros((E_PAD - E,), jnp.int32)])
    dst = jnp.concatenate([edge_index[1],
                           jnp.full((E_PAD - E,), N, jnp.int32)])
    zeros = jnp.zeros((ROWS_PER_TILE, D), jnp.float32)
    batch3 = batch.reshape(NB, 1, BN)

    z = pl.pallas_call(
        _embed_tc,
        grid=(NB,),
        in_specs=[
            pl.BlockSpec((BN, 1), lambda i: (i, 0)),
            pl.BlockSpec((VOCAB_PAD, D), lambda i: (0, 0)),
            pl.BlockSpec((D, D), lambda i: (0, 0)),
        ],
        out_specs=pl.BlockSpec((BN, D), lambda i: (i, 0)),
        out_shape=jax.ShapeDtypeStruct((N, D), jnp.float32),
    )(x, embed_p, Ms[0])

    p1keep = None
    for i in range(L):
        zp = z
        p = _spmm(z, src, dst, zeros)
        if p1keep is None:
            p1keep = p
        if i < L - 1:
            z = pl.pallas_call(
                _combine_tc,
                grid=(NB,),
                in_specs=[
                    pl.BlockSpec((NC, BN, D), lambda j: (0, j, 0)),
                    pl.BlockSpec((D, D), lambda j: (0, 0)),
                    pl.BlockSpec(memory_space=pl.ANY),
                ],
                out_specs=pl.BlockSpec((BN, D), lambda j: (j, 0)),
                out_shape=jax.ShapeDtypeStruct((N, D), jnp.float32),
            )(p, Ms[i + 1], zp)

    out2d = pl.pallas_call(
        _final_tc,
        grid=(NB,),
        in_specs=[
            pl.BlockSpec((NC, BN, D), lambda j: (0, j, 0)),
            pl.BlockSpec((1, 1, BN), lambda j: (j, 0, 0)),
            pl.BlockSpec((D, HID), lambda j: (0, 0)),
            pl.BlockSpec((1, HID), lambda j: (0, 0)),
            pl.BlockSpec((1, HID), lambda j: (0, 0)),
            pl.BlockSpec((1, 1), lambda j: (0, 0)),
            pl.BlockSpec(memory_space=pl.ANY),
            pl.BlockSpec(memory_space=pl.ANY),
            pl.BlockSpec(memory_space=pl.ANY),
            pl.BlockSpec(memory_space=pl.ANY),
            pl.BlockSpec(memory_space=pl.ANY),
        ],
        out_specs=pl.BlockSpec((1, G), lambda j: (0, 0)),
        out_shape=jax.ShapeDtypeStruct((1, G), jnp.float32),
        scratch_shapes=[pltpu.VMEM((G, D), jnp.float32)],
    )(p, batch3, W1, b1.reshape(1, HID), W2.reshape(1, HID), b2.reshape(1, 1),
      zp, src, dst, zeros, p1keep)

    return out2d[0]


# speedup vs baseline: 2.7568x; 1.0005x over previous
import functools

import jax
import jax.numpy as jnp
from jax import lax
from jax.experimental import pallas as pl
from jax.experimental.pallas import tpu as pltpu
from jax.experimental.pallas import tpu_sc as plsc

N = 10000
E = 320000
HID = 32
DIM = 4
D = HID * DIM
L = 3
G = 256
VOCAB = 28
VOCAB_PAD = 32

NC = 2
NS = 16
NW = NC * NS
CHUNK = 80
EPW0 = 17760
EPW1 = 2400
NG0 = EPW0 // (3 * CHUNK)
NG1 = EPW1 // (3 * CHUNK)
OFF1 = NS * EPW0
E_PAD = NS * (EPW0 + EPW1)
N_ACC = 10240
ROWS_PER_TILE = N_ACC // NS

BN = 1000
NB = N // BN


def _spmm_body(z_hbm, src_hbm, dst_hbm, zeros_hbm, out_hbm,
               src_all, d0, d1, d2, r0, r1, r2, acc,
               gs0, gs1, gs2, ds0, ds1, ds2):
    c = lax.axis_index("c")
    s = lax.axis_index("s")
    pltpu.sync_copy(zeros_hbm,
                    acc.at[pl.ds(s * ROWS_PER_TILE, ROWS_PER_TILE)])
    plsc.subcore_barrier()

    rows = (r0, r1, r2)
    dsts = (d0, d1, d2)
    gsems = (gs0, gs1, gs2)
    dsems = (ds0, ds1, ds2)

    def run(ebase, ngroups):
        nch = ngroups * 3

        def gstart(j, k):
            off = pl.multiple_of(j * CHUNK, 8)
            pltpu.async_copy(dst_hbm.at[pl.ds(ebase + off, CHUNK)],
                             dsts[k], dsems[k])
            pltpu.async_copy(z_hbm.at[src_all.at[pl.ds(off, CHUNK)]],
                             rows[k], gsems[k])

        def gwait(k):
            pltpu.make_async_copy(dst_hbm.at[pl.ds(ebase, CHUNK)],
                                  dsts[k], dsems[k]).wait()
            pltpu.make_async_copy(z_hbm.at[src_all.at[pl.ds(0, CHUNK)]],
                                  rows[k], gsems[k]).wait()

        def scat(k):
            pltpu.sync_copy(rows[k], acc.at[dsts[k]], add=True)

        pltpu.sync_copy(src_hbm.at[pl.ds(ebase, nch * CHUNK)],
                        src_all.at[pl.ds(0, nch * CHUNK)])
        for k in range(3):
            gstart(k, k)

        def body(g, carry):
            j = 3 * g
            for k in range(3):
                gwait(k)
                scat(k)
                gstart(j + 3 + k, k)
            return carry

        lax.fori_loop(0, ngroups - 1, body, 0)
        for k in range(3):
            gwait(k)
            scat(k)

    @pl.when(c == 0)
    def _core0():
        run(pl.multiple_of(s * EPW0, 8), NG0)

    @pl.when(c == 1)
    def _core1():
        run(pl.multiple_of(OFF1 + s * EPW1, 8), NG1)

    plsc.subcore_barrier()
    pltpu.sync_copy(acc.at[pl.ds(s * ROWS_PER_TILE, ROWS_PER_TILE)],
                    out_hbm.at[c, pl.ds(s * ROWS_PER_TILE, ROWS_PER_TILE)])


_spmm = functools.partial(
    pl.kernel,
    out_type=jax.ShapeDtypeStruct((NC, N_ACC, D), jnp.float32),
    mesh=plsc.VectorSubcoreMesh(core_axis_name="c", subcore_axis_name="s"),
    scratch_types=[
        pltpu.VMEM((EPW0,), jnp.int32),
        pltpu.VMEM((CHUNK,), jnp.int32),
        pltpu.VMEM((CHUNK,), jnp.int32),
        pltpu.VMEM((CHUNK,), jnp.int32),
        pltpu.VMEM((CHUNK, D), jnp.float32),
        pltpu.VMEM((CHUNK, D), jnp.float32),
        pltpu.VMEM((CHUNK, D), jnp.float32),
        pltpu.VMEM_SHARED((N_ACC, D), jnp.float32),
        pltpu.SemaphoreType.DMA,
        pltpu.SemaphoreType.DMA,
        pltpu.SemaphoreType.DMA,
        pltpu.SemaphoreType.DMA,
        pltpu.SemaphoreType.DMA,
        pltpu.SemaphoreType.DMA,
    ],
)(_spmm_body)


def _embed_tc(x_ref, embed_ref, m_ref, z_ref):
    xb = x_ref[...][:, 0]
    onehot = (xb[:, None]
              == lax.broadcasted_iota(jnp.int32, (BN, VOCAB_PAD), 1)
              ).astype(jnp.float32)
    em = jnp.dot(embed_ref[...], m_ref[...],
                 preferred_element_type=jnp.float32,
                 precision=lax.Precision.HIGHEST)
    z_ref[...] = jnp.dot(onehot, em, preferred_element_type=jnp.float32,
                 precision=lax.Precision.HIGHEST)


def _combine_tc(p_ref, m_ref, d0, z_ref):
    del d0
    h = jnp.maximum(p_ref[0] + p_ref[1], 0.0)
    z_ref[...] = jnp.dot(h, m_ref[...], preferred_element_type=jnp.float32,
                 precision=lax.Precision.HIGHEST)


def _final_tc(p_ref, b_ref, w1_ref, b1_ref, w2t_ref, b2_ref,
              d0, d1, d2, d3, d4, o_ref, y_acc):
    del d0, d1, d2, d3, d4
    i = pl.program_id(0)

    @pl.when(i == 0)
    def _init():
        y_acc[...] = jnp.zeros_like(y_acc)

    h = jnp.maximum(p_ref[0] + p_ref[1], 0.0)
    bb = b_ref[0, 0, :]
    mask = (bb[None, :]
            == lax.broadcasted_iota(jnp.int32, (G, BN), 0)).astype(jnp.float32)
    y_acc[...] += jnp.dot(mask, h, preferred_element_type=jnp.float32,
                 precision=lax.Precision.HIGHEST)

    @pl.when(i == NB - 1)
    def _readout():
        y = y_acc[...]
        t = jnp.maximum(
            jnp.dot(y, w1_ref[...], preferred_element_type=jnp.float32,
                 precision=lax.Precision.HIGHEST)
            + b1_ref[...], 0.0)
        o_ref[...] = (jnp.sum(t * w2t_ref[...], axis=1) + b2_ref[0, 0])[None, :]


def kernel(x, edge_index, batch, embed, Rs, Ws, W1, b1, W2, b2):
    Ms = (jnp.transpose(Rs, (0, 2, 1))[:, :, None, :, None]
          * Ws[:, None, :, None, :]).reshape(L, D, D)
    embed_p = jnp.pad(embed, ((0, VOCAB_PAD - VOCAB), (0, 0)))

    src = jnp.concatenate([edge_index[0],
                           jnp.zeros((E_PAD - E,), jnp.int32)])
    dst = jnp.concatenate([edge_index[1],
                           jnp.full((E_PAD - E,), N, jnp.int32)])
    zeros = jnp.zeros((ROWS_PER_TILE, D), jnp.float32)
    batch3 = batch.reshape(NB, 1, BN)

    z = pl.pallas_call(
        _embed_tc,
        grid=(NB,),
        in_specs=[
            pl.BlockSpec((BN, 1), lambda i: (i, 0)),
            pl.BlockSpec((VOCAB_PAD, D), lambda i: (0, 0)),
            pl.BlockSpec((D, D), lambda i: (0, 0)),
        ],
        out_specs=pl.BlockSpec((BN, D), lambda i: (i, 0)),
        out_shape=jax.ShapeDtypeStruct((N, D), jnp.float32),
    )(x, embed_p, Ms[0])

    p1keep = None
    for i in range(L):
        zp = z
        p = _spmm(z, src, dst, zeros)
        if p1keep is None:
            p1keep = p
        if i < L - 1:
            z = pl.pallas_call(
                _combine_tc,
                grid=(NB,),
                in_specs=[
                    pl.BlockSpec((NC, BN, D), lambda j: (0, j, 0)),
                    pl.BlockSpec((D, D), lambda j: (0, 0)),
                    pl.BlockSpec(memory_space=pl.ANY),
                ],
                out_specs=pl.BlockSpec((BN, D), lambda j: (j, 0)),
                out_shape=jax.ShapeDtypeStruct((N, D), jnp.float32),
            )(p, Ms[i + 1], zp)

    out2d = pl.pallas_call(
        _final_tc,
        grid=(NB,),
        in_specs=[
            pl.BlockSpec((NC, BN, D), lambda j: (0, j, 0)),
            pl.BlockSpec((1, 1, BN), lambda j: (j, 0, 0)),
            pl.BlockSpec((D, HID), lambda j: (0, 0)),
            pl.BlockSpec((1, HID), lambda j: (0, 0)),
            pl.BlockSpec((1, HID), lambda j: (0, 0)),
            pl.BlockSpec((1, 1), lambda j: (0, 0)),
            pl.BlockSpec(memory_space=pl.ANY),
            pl.BlockSpec(memory_space=pl.ANY),
            pl.BlockSpec(memory_space=pl.ANY),
            pl.BlockSpec(memory_space=pl.ANY),
            pl.BlockSpec(memory_space=pl.ANY),
        ],
        out_specs=pl.BlockSpec((1, G), lambda j: (0, 0)),
        out_shape=jax.ShapeDtypeStruct((1, G), jnp.float32),
        scratch_shapes=[pltpu.VMEM((G, D), jnp.float32)],
    )(p, batch3, W1, b1.reshape(1, HID), W2.reshape(1, HID), b2.reshape(1, 1),
      zp, src, dst, zeros, p1keep)

    return out2d[0]
